# baseline (speedup 1.0000x reference)
import jax
import jax.numpy as jnp
from jax import lax
from jax.experimental import pallas as pl
from jax.experimental.pallas import tpu as pltpu


def _gate_slab_kernel(x_ref, w_ref, b_ref, o_ref):
    x = x_ref[0]
    pooled = jnp.sum(x, axis=1, keepdims=True, dtype=jnp.float32)
    y = lax.dot_general(w_ref[...], pooled, (((1,), (0,)), ((), ())),
                        preferred_element_type=jnp.float32)
    g = jax.nn.sigmoid(y + b_ref[...]).astype(o_ref.dtype)
    o_ref[0] = x * g


def kernel(x, weight, bias):
    N, C, T, H, W = x.shape
    S = T * H * W
    if weight.ndim == 5:
        weight = weight.reshape(weight.shape[0], weight.shape[1])

    x_flat = x.reshape(N, C, S)
    w_scaled = weight.astype(jnp.float32) * (1.0 / S)
    b_col = bias.astype(jnp.float32).reshape(C, 1)

    out_flat = pl.pallas_call(
        _gate_slab_kernel,
        out_shape=jax.ShapeDtypeStruct((N, C, S), x.dtype),
        grid=(N,),
        in_specs=[
            pl.BlockSpec((1, C, S), lambda n: (n, 0, 0)),
            pl.BlockSpec((C, C), lambda n: (0, 0)),
            pl.BlockSpec((C, 1), lambda n: (0, 0)),
        ],
        out_specs=pl.BlockSpec((1, C, S), lambda n: (n, 0, 0)),
        compiler_params=pltpu.CompilerParams(
            dimension_semantics=("parallel",),
            vmem_limit_bytes=48 * 1024 * 1024,
        ),
    )(x_flat, w_scaled, b_col)
    return out_flat.reshape(N, C, T, H, W)


# speedup vs baseline: 6.3398x; 6.3398x over previous
import jax
import jax.numpy as jnp
from jax.experimental import pallas as pl
from jax.experimental.pallas import tpu as pltpu


def _gate_nhwtc_kernel(v_ref, w_ref, b_ref, o_ref):
    v = v_ref[0]
    s1 = jnp.sum(v, axis=0, dtype=jnp.float32)
    pooled = jnp.sum(s1, axis=0, keepdims=True)
    y = jnp.dot(pooled, w_ref[...],
                preferred_element_type=jnp.float32) + b_ref[...]
    g = jax.nn.sigmoid(y).astype(o_ref.dtype)
    o_ref[0] = v * g[None]


def kernel(x, weight, bias):
    N, C, T, H, W = x.shape
    S = T * H * W
    HW = H * W
    if weight.ndim == 5:
        weight = weight.reshape(weight.shape[0], weight.shape[1])

    v = jnp.transpose(x, (0, 3, 4, 2, 1)).reshape(N, HW, T, C)
    wt = (weight.astype(jnp.float32) * (1.0 / S)).T
    b_row = bias.astype(jnp.float32).reshape(1, C)

    out_v = pl.pallas_call(
        _gate_nhwtc_kernel,
        out_shape=jax.ShapeDtypeStruct((N, HW, T, C), x.dtype),
        grid=(N,),
        in_specs=[
            pl.BlockSpec((1, HW, T, C), lambda n: (n, 0, 0, 0)),
            pl.BlockSpec((C, C), lambda n: (0, 0)),
            pl.BlockSpec((1, C), lambda n: (0, 0)),
        ],
        out_specs=pl.BlockSpec((1, HW, T, C), lambda n: (n, 0, 0, 0)),
        compiler_params=pltpu.CompilerParams(
            dimension_semantics=("parallel",),
            vmem_limit_bytes=48 * 1024 * 1024,
        ),
    )(v, wt, b_row)

    return jnp.transpose(out_v.reshape(N, H, W, T, C), (0, 4, 3, 1, 2))
